# Initial kernel scaffold; baseline (speedup 1.0000x reference)
import jax
import jax.numpy as jnp
from jax.experimental import pallas as pl


def kernel(inputs, X_train, y_train):
    raise NotImplementedError("write your pallas kernel here")


# speedup vs baseline: 5.1636x; 5.1636x over previous
import jax
import jax.numpy as jnp
from jax.experimental import pallas as pl
from jax.experimental.pallas import tpu as pltpu

_K = 8
_C = 16
_CN = 512


def _body(x_ref, t_ref, y_ref, o_ref, d2_ref, mask_ref):
    x = x_ref[...]
    bq, d = x.shape
    n = d2_ref.shape[1]
    nchunks = n // _CN
    ones_row = jnp.ones((1, d), jnp.float32)

    for c in range(nchunks):
        t = t_ref[pl.ds(c * _CN, _CN), :]
        tn2 = jax.lax.dot_general(
            ones_row, t * t, (((1,), (1,)), ((), ())),
            precision=jax.lax.Precision.HIGHEST,
            preferred_element_type=jnp.float32)
        d2_ref[:, pl.ds(c * _CN, _CN)] = tn2 - 2.0 * jax.lax.dot_general(
            x, t, (((1,), (1,)), ((), ())),
            precision=jax.lax.Precision.HIGHEST,
            preferred_element_type=jnp.float32)
        mask_ref[:, pl.ds(c * _CN, _CN)] = jnp.zeros((bq, _CN), jnp.float32)

    def round_body(_, carry):
        del carry
        m = jnp.full((bq, 1), jnp.inf, jnp.float32)
        first = jnp.full((bq, 1), n, jnp.int32)
        for c in range(nchunks):
            d2c = d2_ref[:, pl.ds(c * _CN, _CN)]
            col = jax.lax.broadcasted_iota(jnp.int32, (bq, _CN), 1) + c * _CN
            cm = jnp.min(d2c, axis=1, keepdims=True)
            cfirst = jnp.min(jnp.where(d2c == cm, col, n), axis=1,
                             keepdims=True)
            better = cm < m
            m = jnp.where(better, cm, m)
            first = jnp.where(better, cfirst, first)
        for c in range(nchunks):
            col = jax.lax.broadcasted_iota(jnp.int32, (bq, _CN), 1) + c * _CN
            sel = col == first
            mask_ref[:, pl.ds(c * _CN, _CN)] = jnp.where(
                sel, 1.0, mask_ref[:, pl.ds(c * _CN, _CN)])
            d2_ref[:, pl.ds(c * _CN, _CN)] = jnp.where(
                sel, jnp.float32(jnp.inf), d2_ref[:, pl.ds(c * _CN, _CN)])
        return 0

    jax.lax.fori_loop(0, _K, round_body, 0)

    counts = jnp.zeros((bq, _C), jnp.float32)
    for c in range(nchunks):
        counts = counts + jax.lax.dot_general(
            mask_ref[:, pl.ds(c * _CN, _CN)], y_ref[pl.ds(c * _CN, _CN), :],
            (((1,), (0,)), ((), ())), preferred_element_type=jnp.float32)
    p = counts * (1.0 / _K)
    r16 = jax.lax.broadcasted_iota(jnp.int32, (_C, _C * _C), 0)
    c256 = jax.lax.broadcasted_iota(jnp.int32, (_C, _C * _C), 1)
    e0 = (c256 == r16 * _C).astype(jnp.float32)
    e1 = (c256 == r16 * _C + 1).astype(jnp.float32)
    o_ref[...] = (
        jax.lax.dot_general(1.0 - p, e0, (((1,), (0,)), ((), ())),
                            preferred_element_type=jnp.float32)
        + jax.lax.dot_general(p, e1, (((1,), (0,)), ((), ())),
                              preferred_element_type=jnp.float32))


def kernel(inputs, X_train, y_train):
    q, d = inputs.shape
    n, c = y_train.shape
    bq = 128
    out2d = pl.pallas_call(
        _body,
        grid=(q // bq,),
        in_specs=[
            pl.BlockSpec((bq, d), lambda i: (i, 0)),
            pl.BlockSpec((n, d), lambda i: (0, 0)),
            pl.BlockSpec((n, c), lambda i: (0, 0)),
        ],
        out_specs=pl.BlockSpec((bq, c * c), lambda i: (i, 0)),
        out_shape=jax.ShapeDtypeStruct((q, c * c), jnp.float32),
        scratch_shapes=[
            pltpu.VMEM((bq, n), jnp.float32),
            pltpu.VMEM((bq, n), jnp.float32),
        ],
    )(inputs, X_train, y_train)
    return out2d.reshape(q, c, c)
